# Initial kernel scaffold; baseline (speedup 1.0000x reference)
import jax
import jax.numpy as jnp
from jax.experimental import pallas as pl


def kernel(x, edge_index, batch, node_emb, conv_W0, conv_b0, conv_W1, conv_b1, conv_W2, conv_b2, fc1_W, fc1_b, fc2_W, fc2_b, out_W, out_b):
    raise NotImplementedError("write your pallas kernel here")


# speedup vs baseline: 7.2442x; 7.2442x over previous
import jax
import jax.numpy as jnp
from jax import lax
from jax.experimental import pallas as pl
from jax.experimental.pallas import tpu as pltpu
from jax.experimental.pallas import tpu_sc as plsc

_N = 50000
_E = 800000
_H = 64
_G = 512
_EPAD = 819200
_NCH = _EPAD // 128
_SROWS = 51200
_TRASH = _N
_ZSPAN = _SROWS // 16
_ZBUF = 800
_BCH = 40

_HIGH = lax.Precision.HIGHEST


def _sc_mesh():
    return plsc.VectorSubcoreMesh(core_axis_name="c", subcore_axis_name="s")


def _edge_kernel(y_hbm, s_hbm, d_hbm, out_hbm, sidx, didx, rows, zbuf, agg,
                 sem):
    c = lax.axis_index("c")
    s = lax.axis_index("s")
    coff = jnp.full((16,), c * _N, jnp.int32)
    z16 = jnp.zeros((16,), jnp.float32)

    def zrow(i, carry):
        zbuf[i, pl.ds(0, 16)] = z16
        return carry
    lax.fori_loop(0, _ZBUF, zrow, 0)

    def zcp(i, carry):
        pltpu.sync_copy(zbuf, agg.at[pl.ds(s * _ZSPAN + i * _ZBUF, _ZBUF)])
        return carry
    lax.fori_loop(0, _ZSPAN // _ZBUF, zcp, 0)
    plsc.subcore_barrier()

    ch0 = s * (_NCH // 16)

    def batch(jb, carry):
        row0 = ch0 + jb * _BCH
        pltpu.sync_copy(s_hbm.at[pl.ds(row0, _BCH)], sidx)
        pltpu.sync_copy(d_hbm.at[pl.ds(row0, _BCH)], didx)

        def adjust(j, carry2):
            for g in range(8):
                sidx[j, pl.ds(g * 16, 16)] = sidx[j, pl.ds(g * 16, 16)] + coff
            return carry2
        lax.fori_loop(0, _BCH, adjust, 0)

        def one(j, carry2):
            pltpu.async_copy(y_hbm.at[sidx.at[j]], rows, sem).wait()
            pltpu.sync_copy(rows, agg.at[didx.at[j]], add=True)
            return carry2
        lax.fori_loop(0, _BCH, one, 0)
        return carry
    lax.fori_loop(0, (_NCH // 16) // _BCH, batch, 0)
    plsc.subcore_barrier()

    base_sp = s * _ZSPAN
    base_out = c * _N + s * _ZSPAN

    @pl.when(s < 15)
    def _():
        def wb(i, carry):
            pltpu.sync_copy(agg.at[pl.ds(base_sp + i * _ZBUF, _ZBUF)], zbuf)
            pltpu.sync_copy(zbuf, out_hbm.at[pl.ds(base_out + i * _ZBUF,
                                                   _ZBUF)])
            return carry
        lax.fori_loop(0, _ZSPAN // _ZBUF, wb, 0)

    @pl.when(s == 15)
    def _():
        def wb(i, carry):
            pltpu.sync_copy(agg.at[pl.ds(base_sp + i * 400, 400)],
                            zbuf.at[pl.ds(0, 400)])
            pltpu.sync_copy(zbuf.at[pl.ds(0, 400)],
                            out_hbm.at[pl.ds(base_out + i * 400, 400)])
            return carry
        lax.fori_loop(0, (_N - 15 * _ZSPAN) // 400, wb, 0)


def _edge_pass(yquart, src2d, dst2d):
    return pl.kernel(
        _edge_kernel,
        out_type=jax.ShapeDtypeStruct((2 * _N, 16), jnp.float32),
        mesh=_sc_mesh(),
        scratch_types=[
            pltpu.VMEM((_BCH, 128), jnp.int32),
            pltpu.VMEM((_BCH, 128), jnp.int32),
            pltpu.VMEM((128, 16), jnp.float32),
            pltpu.VMEM((_ZBUF, 16), jnp.float32),
            pltpu.VMEM_SHARED((_SROWS, 16), jnp.float32),
            pltpu.SemaphoreType.DMA,
        ],
        compiler_params=pltpu.CompilerParams(use_tc_tiling_on_sc=False),
    )(yquart, src2d, dst2d)


_BN = 1000
_NB = _N // _BN


def _quarter_specs():
    return [pl.BlockSpec((_BN, 16), lambda i, q=q: (q * _NB + i, 0))
            for q in range(4)]


def _agg_specs():
    return [pl.BlockSpec((_BN, 16), lambda i, q=q: (q * _NB + i, 0))
            for q in range(2)]


def _cat4(refs):
    return jnp.concatenate([r[...] for r in refs], axis=1)


def _split_quarters(y_ref, y):
    for q in range(4):
        y_ref[q] = y[:, q * 16:(q + 1) * 16]


def _prep_body(x_ref, deg_ref, emb_ref, w_ref, y_ref, dinv_ref):
    deg = deg_ref[:, 0] + 1.0
    dinv = lax.rsqrt(deg)
    dinv_ref[...] = dinv[:, None]
    onehot = (x_ref[...] == lax.broadcasted_iota(jnp.int32, (1, 100), 1))
    ew = jnp.dot(emb_ref[...], w_ref[...], precision=_HIGH)
    y = jnp.dot(onehot.astype(jnp.float32), ew, precision=_HIGH) * dinv[:, None]
    _split_quarters(y_ref, y)


def _prep(x2d, degflat, node_emb, w0):
    return pl.pallas_call(
        _prep_body,
        grid=(_NB,),
        in_specs=[
            pl.BlockSpec((_BN, 1), lambda i: (i, 0)),
            pl.BlockSpec((_BN, 16), lambda i: (i, 0)),
            pl.BlockSpec((100, _H), lambda i: (0, 0)),
            pl.BlockSpec((_H, _H), lambda i: (0, 0)),
        ],
        out_specs=[
            pl.BlockSpec((4, _BN, 16), lambda i: (0, i, 0)),
            pl.BlockSpec((_BN, 1), lambda i: (i, 0)),
        ],
        out_shape=[
            jax.ShapeDtypeStruct((4, _N, 16), jnp.float32),
            jax.ShapeDtypeStruct((_N, 1), jnp.float32),
        ],
    )(x2d, degflat, node_emb, w0)


def _layer_body(a0A, a0B, a1A, a1B, y0, y1, y2, y3, dinv_ref, b_ref, w_ref,
                o_ref):
    dinv = dinv_ref[...]
    agg = _cat4([a0A, a0B, a1A, a1B])
    y = _cat4([y0, y1, y2, y3])
    h = jnp.maximum((agg + y) * dinv + b_ref[...], 0.0)
    yn = jnp.dot(h, w_ref[...], precision=_HIGH) * dinv
    _split_quarters(o_ref, yn)


def _layer(agg_lo, agg_hi, yquart, dinv, b, w_next):
    return pl.pallas_call(
        _layer_body,
        grid=(_NB,),
        in_specs=(
            _agg_specs() + _agg_specs() + _quarter_specs() + [
                pl.BlockSpec((_BN, 1), lambda i: (i, 0)),
                pl.BlockSpec((1, _H), lambda i: (0, 0)),
                pl.BlockSpec((_H, _H), lambda i: (0, 0)),
            ]
        ),
        out_specs=pl.BlockSpec((4, _BN, 16), lambda i: (0, i, 0)),
        out_shape=jax.ShapeDtypeStruct((4, _N, 16), jnp.float32),
    )(agg_lo, agg_lo, agg_hi, agg_hi, yquart, yquart, yquart, yquart,
      dinv, b, w_next)


def _final_body(a0A, a0B, a1A, a1B, y0, y1, y2, y3, dinv_ref, b_ref,
                batch_ref, f1w_ref, f1b_ref, f2w_ref, f2b_ref, ow_ref,
                ob_ref, out_ref, sums, counts):
    i = pl.program_id(0)

    @pl.when(i == 0)
    def _():
        sums[...] = jnp.zeros_like(sums)
        counts[...] = jnp.zeros_like(counts)

    dinv = dinv_ref[...]
    agg = _cat4([a0A, a0B, a1A, a1B])
    y = _cat4([y0, y1, y2, y3])
    h = jnp.maximum((agg + y) * dinv + b_ref[...], 0.0)
    p = (batch_ref[...] == lax.broadcasted_iota(jnp.int32, (1, _G), 1))
    p = p.astype(jnp.float32)
    dn = (((0,), (0,)), ((), ()))
    sums[...] += lax.dot_general(p, h, dn, precision=_HIGH)
    counts[...] += lax.dot_general(p, jnp.ones((_BN, 1), jnp.float32), dn,
                                   precision=_HIGH)

    @pl.when(i == _NB - 1)
    def _():
        pooled = sums[...] / jnp.maximum(counts[...], 1.0)
        a1 = jnp.maximum(jnp.dot(pooled, f1w_ref[...], precision=_HIGH)
                         + f1b_ref[...], 0.0)
        a2 = jnp.maximum(jnp.dot(a1, f2w_ref[...], precision=_HIGH)
                         + f2b_ref[...], 0.0)
        out_ref[...] = jnp.dot(a2, ow_ref[...], precision=_HIGH) + ob_ref[...]


def _final(agg_lo, agg_hi, yquart, dinv, b2, batch2d, f1w, f1b, f2w, f2b,
           ow, ob):
    return pl.pallas_call(
        _final_body,
        grid=(_NB,),
        in_specs=(
            _agg_specs() + _agg_specs() + _quarter_specs() + [
                pl.BlockSpec((_BN, 1), lambda i: (i, 0)),
                pl.BlockSpec((1, _H), lambda i: (0, 0)),
                pl.BlockSpec((_BN, 1), lambda i: (i, 0)),
                pl.BlockSpec((_H, 32), lambda i: (0, 0)),
                pl.BlockSpec((1, 32), lambda i: (0, 0)),
                pl.BlockSpec((32, 16), lambda i: (0, 0)),
                pl.BlockSpec((1, 16), lambda i: (0, 0)),
                pl.BlockSpec((16, 1), lambda i: (0, 0)),
                pl.BlockSpec((1, 1), lambda i: (0, 0)),
            ]
        ),
        out_specs=pl.BlockSpec((_G, 1), lambda i: (0, 0)),
        out_shape=jax.ShapeDtypeStruct((_G, 1), jnp.float32),
        scratch_shapes=[
            pltpu.VMEM((_G, _H), jnp.float32),
            pltpu.VMEM((_G, 1), jnp.float32),
        ],
    )(agg_lo, agg_lo, agg_hi, agg_hi, yquart, yquart, yquart, yquart,
      dinv, b2, batch2d, f1w, f1b, f2w, f2b, ow, ob)


def kernel(x, edge_index, batch, node_emb, conv_W0, conv_b0, conv_W1,
           conv_b1, conv_W2, conv_b2, fc1_W, fc1_b, fc2_W, fc2_b,
           out_W, out_b):
    x2d = x.astype(jnp.int32)
    src = edge_index[0].astype(jnp.int32)
    dst = edge_index[1].astype(jnp.int32)
    pad = _EPAD - _E
    src_p = jnp.concatenate([src, jnp.zeros((pad,), jnp.int32)])
    src_lo = src_p.reshape(_NCH, 128)
    src_hi = (src_p + 2 * _N).reshape(_NCH, 128)
    dst2d = jnp.concatenate(
        [dst, jnp.full((pad,), _TRASH, jnp.int32)]).reshape(_NCH, 128)

    ones_q = jnp.ones((4 * _N, 16), jnp.float32)
    degflat = _edge_pass(ones_q, src_lo, dst2d)
    y, dinv = _prep(x2d, degflat, node_emb, conv_W0)

    b0 = conv_b0.reshape(1, _H)
    b1 = conv_b1.reshape(1, _H)
    b2 = conv_b2.reshape(1, _H)

    yq = y.reshape(4 * _N, 16)
    agg_lo = _edge_pass(yq, src_lo, dst2d)
    agg_hi = _edge_pass(yq, src_hi, dst2d)
    y = _layer(agg_lo, agg_hi, yq, dinv, b0, conv_W1)

    yq = y.reshape(4 * _N, 16)
    agg_lo = _edge_pass(yq, src_lo, dst2d)
    agg_hi = _edge_pass(yq, src_hi, dst2d)
    y = _layer(agg_lo, agg_hi, yq, dinv, b1, conv_W2)

    yq = y.reshape(4 * _N, 16)
    agg_lo = _edge_pass(yq, src_lo, dst2d)
    agg_hi = _edge_pass(yq, src_hi, dst2d)
    out = _final(agg_lo, agg_hi, yq, dinv, b2,
                 batch.reshape(_N, 1).astype(jnp.int32),
                 fc1_W, fc1_b.reshape(1, 32), fc2_W, fc2_b.reshape(1, 16),
                 out_W, out_b.reshape(1, 1))
    return out[:, 0]


# speedup vs baseline: 10.0286x; 1.3844x over previous
import jax
import jax.numpy as jnp
from jax import lax
from jax.experimental import pallas as pl
from jax.experimental.pallas import tpu as pltpu
from jax.experimental.pallas import tpu_sc as plsc

_N = 50000
_E = 800000
_H = 64
_G = 512
_EPAD = 819200
_NCH = _EPAD // 128
_SROWS = 51200
_TRASH = _N
_ZSPAN = _SROWS // 16
_ZBUF = 800
_BCH = 40
_FIRE = 8

_HIGH = lax.Precision.HIGHEST


def _sc_mesh():
    return plsc.VectorSubcoreMesh(core_axis_name="c", subcore_axis_name="s")


def _edge_kernel(y_hbm, s_hbm, d_hbm, out_hbm, sidx, didx, rows, zbuf, agg,
                 sem):
    c = lax.axis_index("c")
    s = lax.axis_index("s")
    z16 = jnp.zeros((16,), jnp.float32)
    ch0 = s * (_NCH // 16)

    def one_pass(p, carry0):
        q = 2 * p + c
        qoff = jnp.full((16,), q * _N, jnp.int32)

        def zrow(i, carry):
            zbuf[i, pl.ds(0, 16)] = z16
            return carry
        lax.fori_loop(0, _ZBUF, zrow, 0)

        def zcp(i, carry):
            pltpu.sync_copy(zbuf, agg.at[pl.ds(s * _ZSPAN + i * _ZBUF,
                                               _ZBUF)])
            return carry
        lax.fori_loop(0, _ZSPAN // _ZBUF, zcp, 0)
        plsc.subcore_barrier()

        def batch(jb, carry):
            row0 = ch0 + jb * _BCH
            pltpu.sync_copy(s_hbm.at[pl.ds(row0, _BCH)], sidx)
            pltpu.sync_copy(d_hbm.at[pl.ds(row0, _BCH)], didx)

            def adjust(j, carry2):
                for g in range(8):
                    sidx[j, pl.ds(g * 16, 16)] = (
                        sidx[j, pl.ds(g * 16, 16)] + qoff)
                return carry2
            lax.fori_loop(0, _BCH, adjust, 0)

            def group(g, carry2):
                g0 = g * _FIRE
                descs = [pltpu.async_copy(y_hbm.at[sidx.at[g0 + k]],
                                          rows.at[k], sem)
                         for k in range(_FIRE)]
                for k in range(_FIRE):
                    descs[k].wait()
                    pltpu.sync_copy(rows.at[k], agg.at[didx.at[g0 + k]],
                                    add=True)
                return carry2
            lax.fori_loop(0, _BCH // _FIRE, group, 0)
            return carry
        lax.fori_loop(0, (_NCH // 16) // _BCH, batch, 0)
        plsc.subcore_barrier()

        base_sp = s * _ZSPAN
        base_out = q * _N + s * _ZSPAN

        @pl.when(s < 15)
        def _():
            def wb(i, carry):
                pltpu.sync_copy(agg.at[pl.ds(base_sp + i * _ZBUF, _ZBUF)],
                                zbuf)
                pltpu.sync_copy(zbuf, out_hbm.at[pl.ds(base_out + i * _ZBUF,
                                                       _ZBUF)])
                return carry
            lax.fori_loop(0, _ZSPAN // _ZBUF, wb, 0)

        @pl.when(s == 15)
        def _():
            def wb(i, carry):
                pltpu.sync_copy(agg.at[pl.ds(base_sp + i * 400, 400)],
                                zbuf.at[pl.ds(0, 400)])
                pltpu.sync_copy(zbuf.at[pl.ds(0, 400)],
                                out_hbm.at[pl.ds(base_out + i * 400, 400)])
                return carry
            lax.fori_loop(0, (_N - 15 * _ZSPAN) // 400, wb, 0)
        return carry0
    lax.fori_loop(0, 2, one_pass, 0)


def _edge_pass(yquart, src2d, dst2d):
    return pl.kernel(
        _edge_kernel,
        out_type=jax.ShapeDtypeStruct((4 * _N, 16), jnp.float32),
        mesh=_sc_mesh(),
        scratch_types=[
            pltpu.VMEM((_BCH, 128), jnp.int32),
            pltpu.VMEM((_BCH, 128), jnp.int32),
            pltpu.VMEM((_FIRE, 128, 16), jnp.float32),
            pltpu.VMEM((_ZBUF, 16), jnp.float32),
            pltpu.VMEM_SHARED((_SROWS, 16), jnp.float32),
            pltpu.SemaphoreType.DMA,
        ],
        compiler_params=pltpu.CompilerParams(use_tc_tiling_on_sc=False),
    )(yquart, src2d, dst2d)


_BN = 1000
_NB = _N // _BN


def _quarter_specs():
    return [pl.BlockSpec((_BN, 16), lambda i, q=q: (q * _NB + i, 0))
            for q in range(4)]


def _cat4(refs):
    return jnp.concatenate([r[...] for r in refs], axis=1)


def _split_quarters(y_ref, y):
    for q in range(4):
        y_ref[q] = y[:, q * 16:(q + 1) * 16]


def _prep_body(x_ref, deg_ref, emb_ref, w_ref, y_ref, dinv_ref):
    deg = deg_ref[:, 0] + 1.0
    dinv = lax.rsqrt(deg)
    dinv_ref[...] = dinv[:, None]
    onehot = (x_ref[...] == lax.broadcasted_iota(jnp.int32, (1, 100), 1))
    ew = jnp.dot(emb_ref[...], w_ref[...], precision=_HIGH)
    y = jnp.dot(onehot.astype(jnp.float32), ew, precision=_HIGH) * dinv[:, None]
    _split_quarters(y_ref, y)


def _prep(x2d, degflat, node_emb, w0):
    return pl.pallas_call(
        _prep_body,
        grid=(_NB,),
        in_specs=[
            pl.BlockSpec((_BN, 1), lambda i: (i, 0)),
            pl.BlockSpec((_BN, 16), lambda i: (i, 0)),
            pl.BlockSpec((100, _H), lambda i: (0, 0)),
            pl.BlockSpec((_H, _H), lambda i: (0, 0)),
        ],
        out_specs=[
            pl.BlockSpec((4, _BN, 16), lambda i: (0, i, 0)),
            pl.BlockSpec((_BN, 1), lambda i: (i, 0)),
        ],
        out_shape=[
            jax.ShapeDtypeStruct((4, _N, 16), jnp.float32),
            jax.ShapeDtypeStruct((_N, 1), jnp.float32),
        ],
    )(x2d, degflat, node_emb, w0)


def _layer_body(a0A, a0B, a1A, a1B, y0, y1, y2, y3, dinv_ref, b_ref, w_ref,
                o_ref):
    dinv = dinv_ref[...]
    agg = _cat4([a0A, a0B, a1A, a1B])
    y = _cat4([y0, y1, y2, y3])
    h = jnp.maximum((agg + y) * dinv + b_ref[...], 0.0)
    yn = jnp.dot(h, w_ref[...], precision=_HIGH) * dinv
    _split_quarters(o_ref, yn)


def _layer(aggquart, yquart, dinv, b, w_next):
    return pl.pallas_call(
        _layer_body,
        grid=(_NB,),
        in_specs=(
            _quarter_specs() + _quarter_specs() + [
                pl.BlockSpec((_BN, 1), lambda i: (i, 0)),
                pl.BlockSpec((1, _H), lambda i: (0, 0)),
                pl.BlockSpec((_H, _H), lambda i: (0, 0)),
            ]
        ),
        out_specs=pl.BlockSpec((4, _BN, 16), lambda i: (0, i, 0)),
        out_shape=jax.ShapeDtypeStruct((4, _N, 16), jnp.float32),
    )(aggquart, aggquart, aggquart, aggquart, yquart, yquart, yquart,
      yquart, dinv, b, w_next)


def _final_body(a0A, a0B, a1A, a1B, y0, y1, y2, y3, dinv_ref, b_ref,
                batch_ref, f1w_ref, f1b_ref, f2w_ref, f2b_ref, ow_ref,
                ob_ref, out_ref, sums, counts):
    i = pl.program_id(0)

    @pl.when(i == 0)
    def _():
        sums[...] = jnp.zeros_like(sums)
        counts[...] = jnp.zeros_like(counts)

    dinv = dinv_ref[...]
    agg = _cat4([a0A, a0B, a1A, a1B])
    y = _cat4([y0, y1, y2, y3])
    h = jnp.maximum((agg + y) * dinv + b_ref[...], 0.0)
    p = (batch_ref[...] == lax.broadcasted_iota(jnp.int32, (1, _G), 1))
    p = p.astype(jnp.float32)
    dn = (((0,), (0,)), ((), ()))
    sums[...] += lax.dot_general(p, h, dn, precision=_HIGH)
    counts[...] += lax.dot_general(p, jnp.ones((_BN, 1), jnp.float32), dn,
                                   precision=_HIGH)

    @pl.when(i == _NB - 1)
    def _():
        pooled = sums[...] / jnp.maximum(counts[...], 1.0)
        a1 = jnp.maximum(jnp.dot(pooled, f1w_ref[...], precision=_HIGH)
                         + f1b_ref[...], 0.0)
        a2 = jnp.maximum(jnp.dot(a1, f2w_ref[...], precision=_HIGH)
                         + f2b_ref[...], 0.0)
        out_ref[...] = jnp.dot(a2, ow_ref[...], precision=_HIGH) + ob_ref[...]


def _final(aggquart, yquart, dinv, b2, batch2d, f1w, f1b, f2w, f2b,
           ow, ob):
    return pl.pallas_call(
        _final_body,
        grid=(_NB,),
        in_specs=(
            _quarter_specs() + _quarter_specs() + [
                pl.BlockSpec((_BN, 1), lambda i: (i, 0)),
                pl.BlockSpec((1, _H), lambda i: (0, 0)),
                pl.BlockSpec((_BN, 1), lambda i: (i, 0)),
                pl.BlockSpec((_H, 32), lambda i: (0, 0)),
                pl.BlockSpec((1, 32), lambda i: (0, 0)),
                pl.BlockSpec((32, 16), lambda i: (0, 0)),
                pl.BlockSpec((1, 16), lambda i: (0, 0)),
                pl.BlockSpec((16, 1), lambda i: (0, 0)),
                pl.BlockSpec((1, 1), lambda i: (0, 0)),
            ]
        ),
        out_specs=pl.BlockSpec((_G, 1), lambda i: (0, 0)),
        out_shape=jax.ShapeDtypeStruct((_G, 1), jnp.float32),
        scratch_shapes=[
            pltpu.VMEM((_G, _H), jnp.float32),
            pltpu.VMEM((_G, 1), jnp.float32),
        ],
    )(aggquart, aggquart, aggquart, aggquart, yquart, yquart, yquart,
      yquart, dinv, b2, batch2d, f1w, f1b, f2w, f2b, ow, ob)


def kernel(x, edge_index, batch, node_emb, conv_W0, conv_b0, conv_W1,
           conv_b1, conv_W2, conv_b2, fc1_W, fc1_b, fc2_W, fc2_b,
           out_W, out_b):
    x2d = x.astype(jnp.int32)
    src = edge_index[0].astype(jnp.int32)
    dst = edge_index[1].astype(jnp.int32)
    pad = _EPAD - _E
    src2d = jnp.concatenate(
        [src, jnp.zeros((pad,), jnp.int32)]).reshape(_NCH, 128)
    dst2d = jnp.concatenate(
        [dst, jnp.full((pad,), _TRASH, jnp.int32)]).reshape(_NCH, 128)

    ones_q = jnp.ones((4 * _N, 16), jnp.float32)
    degflat = _edge_pass(ones_q, src2d, dst2d)
    y, dinv = _prep(x2d, degflat, node_emb, conv_W0)

    b0 = conv_b0.reshape(1, _H)
    b1 = conv_b1.reshape(1, _H)
    b2 = conv_b2.reshape(1, _H)

    yq = y.reshape(4 * _N, 16)
    agg = _edge_pass(yq, src2d, dst2d)
    y = _layer(agg, yq, dinv, b0, conv_W1)

    yq = y.reshape(4 * _N, 16)
    agg = _edge_pass(yq, src2d, dst2d)
    y = _layer(agg, yq, dinv, b1, conv_W2)

    yq = y.reshape(4 * _N, 16)
    agg = _edge_pass(yq, src2d, dst2d)
    out = _final(agg, yq, dinv, b2,
                 batch.reshape(_N, 1).astype(jnp.int32),
                 fc1_W, fc1_b.reshape(1, 32), fc2_W, fc2_b.reshape(1, 16),
                 out_W, out_b.reshape(1, 1))
    return out[:, 0]


# speedup vs baseline: 10.0867x; 1.0058x over previous
import jax
import jax.numpy as jnp
from jax import lax
from jax.experimental import pallas as pl
from jax.experimental.pallas import tpu as pltpu
from jax.experimental.pallas import tpu_sc as plsc

_N = 50000
_E = 800000
_H = 64
_G = 512
_EPAD = 819200
_NCH = _EPAD // 128
_SROWS = 51200
_TRASH = _N
_ZSPAN = _SROWS // 16
_ZBUF = 800
_BCH = 40
_FIRE = 8

_HIGH = lax.Precision.HIGHEST


def _sc_mesh():
    return plsc.VectorSubcoreMesh(core_axis_name="c", subcore_axis_name="s")


def _edge_kernel(y_hbm, s_hbm, d_hbm, out_hbm, sidx, didx, rows, zbuf, agg,
                 sem):
    c = lax.axis_index("c")
    s = lax.axis_index("s")
    z16 = jnp.zeros((16,), jnp.float32)
    ch0 = s * (_NCH // 16)

    def one_pass(p, carry0):
        q = 2 * p + c
        qoff = jnp.full((16,), q * _N, jnp.int32)

        def zrow(i, carry):
            zbuf[i, pl.ds(0, 16)] = z16
            return carry
        lax.fori_loop(0, _ZBUF, zrow, 0)

        def zcp(i, carry):
            pltpu.sync_copy(zbuf, agg.at[pl.ds(s * _ZSPAN + i * _ZBUF,
                                               _ZBUF)])
            return carry
        lax.fori_loop(0, _ZSPAN // _ZBUF, zcp, 0)
        plsc.subcore_barrier()

        def batch(jb, carry):
            row0 = ch0 + jb * _BCH
            pltpu.sync_copy(s_hbm.at[pl.ds(row0, _BCH)], sidx)
            pltpu.sync_copy(d_hbm.at[pl.ds(row0, _BCH)], didx)

            def adjust(j, carry2):
                for g in range(8):
                    sidx[j, pl.ds(g * 16, 16)] = (
                        sidx[j, pl.ds(g * 16, 16)] + qoff)
                return carry2
            lax.fori_loop(0, _BCH, adjust, 0)

            def group(g, carry2):
                g0 = g * _FIRE
                descs = [pltpu.async_copy(y_hbm.at[sidx.at[g0 + k]],
                                          rows.at[k], sem)
                         for k in range(_FIRE)]
                for k in range(_FIRE):
                    descs[k].wait()
                    pltpu.sync_copy(rows.at[k], agg.at[didx.at[g0 + k]],
                                    add=True)
                return carry2
            lax.fori_loop(0, _BCH // _FIRE, group, 0)
            return carry
        lax.fori_loop(0, (_NCH // 16) // _BCH, batch, 0)
        plsc.subcore_barrier()

        base_sp = s * _ZSPAN
        base_out = q * _N + s * _ZSPAN

        @pl.when(s < 15)
        def _():
            def wb(i, carry):
                pltpu.sync_copy(agg.at[pl.ds(base_sp + i * _ZBUF, _ZBUF)],
                                zbuf)
                pltpu.sync_copy(zbuf, out_hbm.at[pl.ds(base_out + i * _ZBUF,
                                                       _ZBUF)])
                return carry
            lax.fori_loop(0, _ZSPAN // _ZBUF, wb, 0)

        @pl.when(s == 15)
        def _():
            def wb(i, carry):
                pltpu.sync_copy(agg.at[pl.ds(base_sp + i * 400, 400)],
                                zbuf.at[pl.ds(0, 400)])
                pltpu.sync_copy(zbuf.at[pl.ds(0, 400)],
                                out_hbm.at[pl.ds(base_out + i * 400, 400)])
                return carry
            lax.fori_loop(0, (_N - 15 * _ZSPAN) // 400, wb, 0)
        return carry0
    lax.fori_loop(0, 2, one_pass, 0)


def _edge_pass(yquart, src2d, dst2d):
    return pl.kernel(
        _edge_kernel,
        out_type=jax.ShapeDtypeStruct((4 * _N, 16), jnp.float32),
        mesh=_sc_mesh(),
        scratch_types=[
            pltpu.VMEM((_BCH, 128), jnp.int32),
            pltpu.VMEM((_BCH, 128), jnp.int32),
            pltpu.VMEM((_FIRE, 128, 16), jnp.float32),
            pltpu.VMEM((_ZBUF, 16), jnp.float32),
            pltpu.VMEM_SHARED((_SROWS, 16), jnp.float32),
            pltpu.SemaphoreType.DMA,
        ],
        compiler_params=pltpu.CompilerParams(use_tc_tiling_on_sc=False),
    )(yquart, src2d, dst2d)


_BN = 1000
_NB = _N // _BN


def _quarter_specs():
    return [pl.BlockSpec((_BN, 16), lambda i, q=q: (q * _NB + i, 0))
            for q in range(4)]


def _cat4(refs):
    return jnp.concatenate([r[...] for r in refs], axis=1)


def _split_quarters(y_ref, y):
    for q in range(4):
        y_ref[q] = y[:, q * 16:(q + 1) * 16]


def _prep_body(x_ref, deg_ref, emb_ref, w_ref, y_ref, dinv_ref):
    deg = deg_ref[:, 0] + 1.0
    dinv = 1.0 / jnp.sqrt(deg)
    dinv_ref[...] = dinv[:, None]
    onehot = (x_ref[...] == lax.broadcasted_iota(jnp.int32, (1, 100), 1))
    h0 = jnp.dot(onehot.astype(jnp.float32), emb_ref[...], precision=_HIGH)
    y = jnp.dot(h0, w_ref[...]) * dinv[:, None]
    _split_quarters(y_ref, y)


def _prep(x2d, degflat, node_emb, w0):
    return pl.pallas_call(
        _prep_body,
        grid=(_NB,),
        in_specs=[
            pl.BlockSpec((_BN, 1), lambda i: (i, 0)),
            pl.BlockSpec((_BN, 16), lambda i: (i, 0)),
            pl.BlockSpec((100, _H), lambda i: (0, 0)),
            pl.BlockSpec((_H, _H), lambda i: (0, 0)),
        ],
        out_specs=[
            pl.BlockSpec((4, _BN, 16), lambda i: (0, i, 0)),
            pl.BlockSpec((_BN, 1), lambda i: (i, 0)),
        ],
        out_shape=[
            jax.ShapeDtypeStruct((4, _N, 16), jnp.float32),
            jax.ShapeDtypeStruct((_N, 1), jnp.float32),
        ],
    )(x2d, degflat, node_emb, w0)


def _layer_body(a0A, a0B, a1A, a1B, y0, y1, y2, y3, dinv_ref, b_ref, w_ref,
                o_ref):
    dinv = dinv_ref[...]
    agg = _cat4([a0A, a0B, a1A, a1B])
    y = _cat4([y0, y1, y2, y3])
    h = jnp.maximum((agg + y) * dinv + b_ref[...], 0.0)
    yn = jnp.dot(h, w_ref[...]) * dinv
    _split_quarters(o_ref, yn)


def _layer(aggquart, yquart, dinv, b, w_next):
    return pl.pallas_call(
        _layer_body,
        grid=(_NB,),
        in_specs=(
            _quarter_specs() + _quarter_specs() + [
                pl.BlockSpec((_BN, 1), lambda i: (i, 0)),
                pl.BlockSpec((1, _H), lambda i: (0, 0)),
                pl.BlockSpec((_H, _H), lambda i: (0, 0)),
            ]
        ),
        out_specs=pl.BlockSpec((4, _BN, 16), lambda i: (0, i, 0)),
        out_shape=jax.ShapeDtypeStruct((4, _N, 16), jnp.float32),
    )(aggquart, aggquart, aggquart, aggquart, yquart, yquart, yquart,
      yquart, dinv, b, w_next)


def _final_body(a0A, a0B, a1A, a1B, y0, y1, y2, y3, dinv_ref, b_ref,
                batch_ref, f1w_ref, f1b_ref, f2w_ref, f2b_ref, ow_ref,
                ob_ref, out_ref, sums, counts):
    i = pl.program_id(0)

    @pl.when(i == 0)
    def _():
        sums[...] = jnp.zeros_like(sums)
        counts[...] = jnp.zeros_like(counts)

    dinv = dinv_ref[...]
    agg = _cat4([a0A, a0B, a1A, a1B])
    y = _cat4([y0, y1, y2, y3])
    h = jnp.maximum((agg + y) * dinv + b_ref[...], 0.0)
    p = (batch_ref[...] == lax.broadcasted_iota(jnp.int32, (1, _G), 1))
    p = p.astype(jnp.float32)
    dn = (((0,), (0,)), ((), ()))
    sums[...] += lax.dot_general(p, h, dn, precision=_HIGH)
    counts[...] += lax.dot_general(p, jnp.ones((_BN, 1), jnp.float32), dn,
                                   precision=_HIGH)

    @pl.when(i == _NB - 1)
    def _():
        pooled = sums[...] / jnp.maximum(counts[...], 1.0)
        a1 = jnp.maximum(jnp.dot(pooled, f1w_ref[...]) + f1b_ref[...], 0.0)
        a2 = jnp.maximum(jnp.dot(a1, f2w_ref[...]) + f2b_ref[...], 0.0)
        out_ref[...] = jnp.dot(a2, ow_ref[...]) + ob_ref[...]


def _final(aggquart, yquart, dinv, b2, batch2d, f1w, f1b, f2w, f2b,
           ow, ob):
    return pl.pallas_call(
        _final_body,
        grid=(_NB,),
        in_specs=(
            _quarter_specs() + _quarter_specs() + [
                pl.BlockSpec((_BN, 1), lambda i: (i, 0)),
                pl.BlockSpec((1, _H), lambda i: (0, 0)),
                pl.BlockSpec((_BN, 1), lambda i: (i, 0)),
                pl.BlockSpec((_H, 32), lambda i: (0, 0)),
                pl.BlockSpec((1, 32), lambda i: (0, 0)),
                pl.BlockSpec((32, 16), lambda i: (0, 0)),
                pl.BlockSpec((1, 16), lambda i: (0, 0)),
                pl.BlockSpec((16, 1), lambda i: (0, 0)),
                pl.BlockSpec((1, 1), lambda i: (0, 0)),
            ]
        ),
        out_specs=pl.BlockSpec((_G, 1), lambda i: (0, 0)),
        out_shape=jax.ShapeDtypeStruct((_G, 1), jnp.float32),
        scratch_shapes=[
            pltpu.VMEM((_G, _H), jnp.float32),
            pltpu.VMEM((_G, 1), jnp.float32),
        ],
    )(aggquart, aggquart, aggquart, aggquart, yquart, yquart, yquart,
      yquart, dinv, b2, batch2d, f1w, f1b, f2w, f2b, ow, ob)


def kernel(x, edge_index, batch, node_emb, conv_W0, conv_b0, conv_W1,
           conv_b1, conv_W2, conv_b2, fc1_W, fc1_b, fc2_W, fc2_b,
           out_W, out_b):
    x2d = x.astype(jnp.int32)
    src = edge_index[0].astype(jnp.int32)
    dst = edge_index[1].astype(jnp.int32)
    pad = _EPAD - _E
    src2d = jnp.concatenate(
        [src, jnp.zeros((pad,), jnp.int32)]).reshape(_NCH, 128)
    dst2d = jnp.concatenate(
        [dst, jnp.full((pad,), _TRASH, jnp.int32)]).reshape(_NCH, 128)

    ones_q = jnp.ones((4 * _N, 16), jnp.float32)
    degflat = _edge_pass(ones_q, src2d, dst2d)
    y, dinv = _prep(x2d, degflat, node_emb, conv_W0)

    b0 = conv_b0.reshape(1, _H)
    b1 = conv_b1.reshape(1, _H)
    b2 = conv_b2.reshape(1, _H)

    yq = y.reshape(4 * _N, 16)
    agg = _edge_pass(yq, src2d, dst2d)
    y = _layer(agg, yq, dinv, b0, conv_W1)

    yq = y.reshape(4 * _N, 16)
    agg = _edge_pass(yq, src2d, dst2d)
    y = _layer(agg, yq, dinv, b1, conv_W2)

    yq = y.reshape(4 * _N, 16)
    agg = _edge_pass(yq, src2d, dst2d)
    out = _final(agg, yq, dinv, b2,
                 batch.reshape(_N, 1).astype(jnp.int32),
                 fc1_W, fc1_b.reshape(1, 32), fc2_W, fc2_b.reshape(1, 16),
                 out_W, out_b.reshape(1, 1))
    return out[:, 0]


# speedup vs baseline: 10.5744x; 1.0484x over previous
import jax
import jax.numpy as jnp
from jax import lax
from jax.experimental import pallas as pl
from jax.experimental.pallas import tpu as pltpu
from jax.experimental.pallas import tpu_sc as plsc

_N = 50000
_E = 800000
_H = 64
_G = 512
_EPAD = 819200
_NCH = _EPAD // 128
_SROWS = 51200
_TRASH = _N
_ZSPAN = _SROWS // 16
_ZBUF = 800
_BCH = 40
_FIRE = 8

_HIGH = lax.Precision.HIGHEST


def _sc_mesh():
    return plsc.VectorSubcoreMesh(core_axis_name="c", subcore_axis_name="s")


def _edge_kernel(y_hbm, s_hbm, d_hbm, out_hbm, sidx, didx, rows, zbuf, agg,
                 sem, sem2):
    c = lax.axis_index("c")
    s = lax.axis_index("s")
    z16 = jnp.zeros((16,), jnp.float32)
    ch0 = s * (_NCH // 16)

    def one_pass(p, carry0):
        q = 2 * p + c
        qoff = jnp.full((16,), q * _N, jnp.int32)

        def zrow(i, carry):
            zbuf[i, pl.ds(0, 16)] = z16
            return carry
        lax.fori_loop(0, _ZBUF, zrow, 0)

        def zcp(i, carry):
            pltpu.sync_copy(zbuf, agg.at[pl.ds(s * _ZSPAN + i * _ZBUF,
                                               _ZBUF)])
            return carry
        lax.fori_loop(0, _ZSPAN // _ZBUF, zcp, 0)
        plsc.subcore_barrier()

        def batch(jb, carry):
            row0 = ch0 + jb * _BCH
            pltpu.sync_copy(s_hbm.at[pl.ds(row0, _BCH)], sidx)
            pltpu.sync_copy(d_hbm.at[pl.ds(row0, _BCH)], didx)

            def adjust(j, carry2):
                for g in range(8):
                    sidx[j, pl.ds(g * 16, 16)] = (
                        sidx[j, pl.ds(g * 16, 16)] + qoff)
                return carry2
            lax.fori_loop(0, _BCH, adjust, 0)

            gd = [None] * _BCH
            sd = [None] * _BCH
            lag = _FIRE // 2
            for j in range(_BCH + lag):
                if j < _BCH:
                    if j >= _FIRE:
                        sd[j - _FIRE].wait()
                    gd[j] = pltpu.async_copy(y_hbm.at[sidx.at[j]],
                                             rows.at[j % _FIRE], sem)
                if j >= lag and j - lag < _BCH:
                    k = j - lag
                    gd[k].wait()
                    sd[k] = pltpu.async_copy(rows.at[k % _FIRE],
                                             agg.at[didx.at[k]], sem2,
                                             add=True)
            for k in range(_BCH - _FIRE, _BCH):
                sd[k].wait()
            return carry
        lax.fori_loop(0, (_NCH // 16) // _BCH, batch, 0)
        plsc.subcore_barrier()

        base_sp = s * _ZSPAN
        base_out = q * _N + s * _ZSPAN

        @pl.when(s < 15)
        def _():
            def wb(i, carry):
                pltpu.sync_copy(agg.at[pl.ds(base_sp + i * _ZBUF, _ZBUF)],
                                zbuf)
                pltpu.sync_copy(zbuf, out_hbm.at[pl.ds(base_out + i * _ZBUF,
                                                       _ZBUF)])
                return carry
            lax.fori_loop(0, _ZSPAN // _ZBUF, wb, 0)

        @pl.when(s == 15)
        def _():
            def wb(i, carry):
                pltpu.sync_copy(agg.at[pl.ds(base_sp + i * 400, 400)],
                                zbuf.at[pl.ds(0, 400)])
                pltpu.sync_copy(zbuf.at[pl.ds(0, 400)],
                                out_hbm.at[pl.ds(base_out + i * 400, 400)])
                return carry
            lax.fori_loop(0, (_N - 15 * _ZSPAN) // 400, wb, 0)
        return carry0
    lax.fori_loop(0, 2, one_pass, 0)


def _edge_pass(yquart, src2d, dst2d):
    return pl.kernel(
        _edge_kernel,
        out_type=jax.ShapeDtypeStruct((4 * _N, 16), jnp.float32),
        mesh=_sc_mesh(),
        scratch_types=[
            pltpu.VMEM((_BCH, 128), jnp.int32),
            pltpu.VMEM((_BCH, 128), jnp.int32),
            pltpu.VMEM((_FIRE, 128, 16), jnp.float32),
            pltpu.VMEM((_ZBUF, 16), jnp.float32),
            pltpu.VMEM_SHARED((_SROWS, 16), jnp.float32),
            pltpu.SemaphoreType.DMA,
            pltpu.SemaphoreType.DMA,
        ],
        compiler_params=pltpu.CompilerParams(use_tc_tiling_on_sc=False),
    )(yquart, src2d, dst2d)


_BN = 1000
_NB = _N // _BN


def _quarter_specs():
    return [pl.BlockSpec((_BN, 16), lambda i, q=q: (q * _NB + i, 0))
            for q in range(4)]


def _cat4(refs):
    return jnp.concatenate([r[...] for r in refs], axis=1)


def _split_quarters(y_ref, y):
    for q in range(4):
        y_ref[q] = y[:, q * 16:(q + 1) * 16]


def _prep_body(x_ref, deg_ref, emb_ref, w_ref, y_ref, dinv_ref):
    deg = deg_ref[:, 0] + 1.0
    dinv = 1.0 / jnp.sqrt(deg)
    dinv_ref[...] = dinv[:, None]
    onehot = (x_ref[...] == lax.broadcasted_iota(jnp.int32, (1, 100), 1))
    h0 = jnp.dot(onehot.astype(jnp.float32), emb_ref[...], precision=_HIGH)
    y = jnp.dot(h0, w_ref[...]) * dinv[:, None]
    _split_quarters(y_ref, y)


def _prep(x2d, degflat, node_emb, w0):
    return pl.pallas_call(
        _prep_body,
        grid=(_NB,),
        in_specs=[
            pl.BlockSpec((_BN, 1), lambda i: (i, 0)),
            pl.BlockSpec((_BN, 16), lambda i: (i, 0)),
            pl.BlockSpec((100, _H), lambda i: (0, 0)),
            pl.BlockSpec((_H, _H), lambda i: (0, 0)),
        ],
        out_specs=[
            pl.BlockSpec((4, _BN, 16), lambda i: (0, i, 0)),
            pl.BlockSpec((_BN, 1), lambda i: (i, 0)),
        ],
        out_shape=[
            jax.ShapeDtypeStruct((4, _N, 16), jnp.float32),
            jax.ShapeDtypeStruct((_N, 1), jnp.float32),
        ],
    )(x2d, degflat, node_emb, w0)


def _layer_body(a0A, a0B, a1A, a1B, y0, y1, y2, y3, dinv_ref, b_ref, w_ref,
                o_ref):
    dinv = dinv_ref[...]
    agg = _cat4([a0A, a0B, a1A, a1B])
    y = _cat4([y0, y1, y2, y3])
    h = jnp.maximum((agg + y) * dinv + b_ref[...], 0.0)
    yn = jnp.dot(h, w_ref[...]) * dinv
    _split_quarters(o_ref, yn)


def _layer(aggquart, yquart, dinv, b, w_next):
    return pl.pallas_call(
        _layer_body,
        grid=(_NB,),
        in_specs=(
            _quarter_specs() + _quarter_specs() + [
                pl.BlockSpec((_BN, 1), lambda i: (i, 0)),
                pl.BlockSpec((1, _H), lambda i: (0, 0)),
                pl.BlockSpec((_H, _H), lambda i: (0, 0)),
            ]
        ),
        out_specs=pl.BlockSpec((4, _BN, 16), lambda i: (0, i, 0)),
        out_shape=jax.ShapeDtypeStruct((4, _N, 16), jnp.float32),
    )(aggquart, aggquart, aggquart, aggquart, yquart, yquart, yquart,
      yquart, dinv, b, w_next)


def _final_body(a0A, a0B, a1A, a1B, y0, y1, y2, y3, dinv_ref, b_ref,
                batch_ref, f1w_ref, f1b_ref, f2w_ref, f2b_ref, ow_ref,
                ob_ref, out_ref, sums, counts):
    i = pl.program_id(0)

    @pl.when(i == 0)
    def _():
        sums[...] = jnp.zeros_like(sums)
        counts[...] = jnp.zeros_like(counts)

    dinv = dinv_ref[...]
    agg = _cat4([a0A, a0B, a1A, a1B])
    y = _cat4([y0, y1, y2, y3])
    h = jnp.maximum((agg + y) * dinv + b_ref[...], 0.0)
    p = (batch_ref[...] == lax.broadcasted_iota(jnp.int32, (1, _G), 1))
    p = p.astype(jnp.float32)
    dn = (((0,), (0,)), ((), ()))
    sums[...] += lax.dot_general(p, h, dn, precision=_HIGH)
    counts[...] += lax.dot_general(p, jnp.ones((_BN, 1), jnp.float32), dn,
                                   precision=_HIGH)

    @pl.when(i == _NB - 1)
    def _():
        pooled = sums[...] / jnp.maximum(counts[...], 1.0)
        a1 = jnp.maximum(jnp.dot(pooled, f1w_ref[...]) + f1b_ref[...], 0.0)
        a2 = jnp.maximum(jnp.dot(a1, f2w_ref[...]) + f2b_ref[...], 0.0)
        out_ref[...] = jnp.dot(a2, ow_ref[...]) + ob_ref[...]


def _final(aggquart, yquart, dinv, b2, batch2d, f1w, f1b, f2w, f2b,
           ow, ob):
    return pl.pallas_call(
        _final_body,
        grid=(_NB,),
        in_specs=(
            _quarter_specs() + _quarter_specs() + [
                pl.BlockSpec((_BN, 1), lambda i: (i, 0)),
                pl.BlockSpec((1, _H), lambda i: (0, 0)),
                pl.BlockSpec((_BN, 1), lambda i: (i, 0)),
                pl.BlockSpec((_H, 32), lambda i: (0, 0)),
                pl.BlockSpec((1, 32), lambda i: (0, 0)),
                pl.BlockSpec((32, 16), lambda i: (0, 0)),
                pl.BlockSpec((1, 16), lambda i: (0, 0)),
                pl.BlockSpec((16, 1), lambda i: (0, 0)),
                pl.BlockSpec((1, 1), lambda i: (0, 0)),
            ]
        ),
        out_specs=pl.BlockSpec((_G, 1), lambda i: (0, 0)),
        out_shape=jax.ShapeDtypeStruct((_G, 1), jnp.float32),
        scratch_shapes=[
            pltpu.VMEM((_G, _H), jnp.float32),
            pltpu.VMEM((_G, 1), jnp.float32),
        ],
    )(aggquart, aggquart, aggquart, aggquart, yquart, yquart, yquart,
      yquart, dinv, b2, batch2d, f1w, f1b, f2w, f2b, ow, ob)


def kernel(x, edge_index, batch, node_emb, conv_W0, conv_b0, conv_W1,
           conv_b1, conv_W2, conv_b2, fc1_W, fc1_b, fc2_W, fc2_b,
           out_W, out_b):
    x2d = x.astype(jnp.int32)
    src = edge_index[0].astype(jnp.int32)
    dst = edge_index[1].astype(jnp.int32)
    pad = _EPAD - _E
    src2d = jnp.concatenate(
        [src, jnp.zeros((pad,), jnp.int32)]).reshape(_NCH, 128)
    dst2d = jnp.concatenate(
        [dst, jnp.full((pad,), _TRASH, jnp.int32)]).reshape(_NCH, 128)

    ones_q = jnp.ones((4 * _N, 16), jnp.float32)
    degflat = _edge_pass(ones_q, src2d, dst2d)
    y, dinv = _prep(x2d, degflat, node_emb, conv_W0)

    b0 = conv_b0.reshape(1, _H)
    b1 = conv_b1.reshape(1, _H)
    b2 = conv_b2.reshape(1, _H)

    yq = y.reshape(4 * _N, 16)
    agg = _edge_pass(yq, src2d, dst2d)
    y = _layer(agg, yq, dinv, b0, conv_W1)

    yq = y.reshape(4 * _N, 16)
    agg = _edge_pass(yq, src2d, dst2d)
    y = _layer(agg, yq, dinv, b1, conv_W2)

    yq = y.reshape(4 * _N, 16)
    agg = _edge_pass(yq, src2d, dst2d)
    out = _final(agg, yq, dinv, b2,
                 batch.reshape(_N, 1).astype(jnp.int32),
                 fc1_W, fc1_b.reshape(1, 32), fc2_W, fc2_b.reshape(1, 16),
                 out_W, out_b.reshape(1, 1))
    return out[:, 0]


# speedup vs baseline: 11.3732x; 1.0755x over previous
import jax
import jax.numpy as jnp
from jax import lax
from jax.experimental import pallas as pl
from jax.experimental.pallas import tpu as pltpu
from jax.experimental.pallas import tpu_sc as plsc

_N = 50000
_E = 800000
_H = 64
_G = 512
_EPAD = 819200
_NCH = _EPAD // 128
_SROWS = 51200
_TRASH = _N
_ZSPAN = _SROWS // 16
_ZBUF = 800
_BCH = 40
_FIRE = 8

_HIGH = lax.Precision.HIGHEST


def _sc_mesh():
    return plsc.VectorSubcoreMesh(core_axis_name="c", subcore_axis_name="s")


def _edge_kernel(y_hbm, s_hbm, d_hbm, np_hbm, out_hbm, sidx, didx, rows,
                 zbuf, npv, agg, sem, sem2):
    c = lax.axis_index("c")
    s = lax.axis_index("s")
    z16 = jnp.zeros((16,), jnp.float32)
    ch0 = s * (_NCH // 16)
    pltpu.sync_copy(np_hbm, npv)
    npasses = npv[...][0]

    def one_pass(p, carry0):
        q = 2 * p + c
        qoff = jnp.full((16,), q * _N, jnp.int32)

        def zrow(i, carry):
            zbuf[i, pl.ds(0, 16)] = z16
            return carry
        lax.fori_loop(0, _ZBUF, zrow, 0)

        def zcp(i, carry):
            pltpu.sync_copy(zbuf, agg.at[pl.ds(s * _ZSPAN + i * _ZBUF,
                                               _ZBUF)])
            return carry
        lax.fori_loop(0, _ZSPAN // _ZBUF, zcp, 0)
        plsc.subcore_barrier()

        def batch(jb, carry):
            row0 = ch0 + jb * _BCH
            pltpu.sync_copy(s_hbm.at[pl.ds(row0, _BCH)], sidx)
            pltpu.sync_copy(d_hbm.at[pl.ds(row0, _BCH)], didx)

            def adjust(j, carry2):
                for g in range(8):
                    sidx[j, pl.ds(g * 16, 16)] = (
                        sidx[j, pl.ds(g * 16, 16)] + qoff)
                return carry2
            lax.fori_loop(0, _BCH, adjust, 0)

            gd = [None] * _BCH
            sd = [None] * _BCH
            lag = _FIRE // 2
            for j in range(_BCH + lag):
                if j < _BCH:
                    if j >= _FIRE:
                        sd[j - _FIRE].wait()
                    gd[j] = pltpu.async_copy(y_hbm.at[sidx.at[j]],
                                             rows.at[j % _FIRE], sem)
                if j >= lag and j - lag < _BCH:
                    k = j - lag
                    gd[k].wait()
                    sd[k] = pltpu.async_copy(rows.at[k % _FIRE],
                                             agg.at[didx.at[k]], sem2,
                                             add=True)
            for k in range(_BCH - _FIRE, _BCH):
                sd[k].wait()
            return carry
        lax.fori_loop(0, (_NCH // 16) // _BCH, batch, 0)
        plsc.subcore_barrier()

        base_sp = s * _ZSPAN
        base_out = q * _N + s * _ZSPAN

        @pl.when(s < 15)
        def _():
            def wb(i, carry):
                pltpu.sync_copy(agg.at[pl.ds(base_sp + i * _ZBUF, _ZBUF)],
                                zbuf)
                pltpu.sync_copy(zbuf, out_hbm.at[pl.ds(base_out + i * _ZBUF,
                                                       _ZBUF)])
                return carry
            lax.fori_loop(0, _ZSPAN // _ZBUF, wb, 0)

        @pl.when(s == 15)
        def _():
            def wb(i, carry):
                pltpu.sync_copy(agg.at[pl.ds(base_sp + i * 400, 400)],
                                zbuf.at[pl.ds(0, 400)])
                pltpu.sync_copy(zbuf.at[pl.ds(0, 400)],
                                out_hbm.at[pl.ds(base_out + i * 400, 400)])
                return carry
            lax.fori_loop(0, (_N - 15 * _ZSPAN) // 400, wb, 0)
        return carry0
    lax.fori_loop(0, npasses, one_pass, 0)


def _edge_pass(yquart, src2d, dst2d, npasses):
    return pl.kernel(
        _edge_kernel,
        out_type=jax.ShapeDtypeStruct((4 * _N, 16), jnp.float32),
        mesh=_sc_mesh(),
        scratch_types=[
            pltpu.VMEM((_BCH, 128), jnp.int32),
            pltpu.VMEM((_BCH, 128), jnp.int32),
            pltpu.VMEM((_FIRE, 128, 16), jnp.float32),
            pltpu.VMEM((_ZBUF, 16), jnp.float32),
            pltpu.VMEM((16,), jnp.int32),
            pltpu.VMEM_SHARED((_SROWS, 16), jnp.float32),
            pltpu.SemaphoreType.DMA,
            pltpu.SemaphoreType.DMA,
        ],
        compiler_params=pltpu.CompilerParams(use_tc_tiling_on_sc=False),
    )(yquart, src2d, dst2d, npasses)


_BN = 1000
_NB = _N // _BN


def _quarter_specs():
    return [pl.BlockSpec((_BN, 16), lambda i, q=q: (q * _NB + i, 0))
            for q in range(4)]


def _cat4(refs):
    return jnp.concatenate([r[...] for r in refs], axis=1)


def _split_quarters(y_ref, y):
    for q in range(4):
        y_ref[q] = y[:, q * 16:(q + 1) * 16]


def _prep_body(x_ref, deg_ref, emb_ref, w_ref, y_ref, dinv_ref):
    deg = deg_ref[:, 0] + 1.0
    dinv = 1.0 / jnp.sqrt(deg)
    dinv_ref[...] = dinv[:, None]
    onehot = (x_ref[...] == lax.broadcasted_iota(jnp.int32, (1, 100), 1))
    h0 = jnp.dot(onehot.astype(jnp.float32), emb_ref[...], precision=_HIGH)
    y = jnp.dot(h0, w_ref[...]) * dinv[:, None]
    _split_quarters(y_ref, y)


def _prep(x2d, degflat, node_emb, w0):
    return pl.pallas_call(
        _prep_body,
        grid=(_NB,),
        in_specs=[
            pl.BlockSpec((_BN, 1), lambda i: (i, 0)),
            pl.BlockSpec((_BN, 16), lambda i: (i, 0)),
            pl.BlockSpec((100, _H), lambda i: (0, 0)),
            pl.BlockSpec((_H, _H), lambda i: (0, 0)),
        ],
        out_specs=[
            pl.BlockSpec((4, _BN, 16), lambda i: (0, i, 0)),
            pl.BlockSpec((_BN, 1), lambda i: (i, 0)),
        ],
        out_shape=[
            jax.ShapeDtypeStruct((4, _N, 16), jnp.float32),
            jax.ShapeDtypeStruct((_N, 1), jnp.float32),
        ],
    )(x2d, degflat, node_emb, w0)


def _layer_body(a0A, a0B, a1A, a1B, y0, y1, y2, y3, dinv_ref, b_ref, w_ref,
                o_ref):
    dinv = dinv_ref[...]
    agg = _cat4([a0A, a0B, a1A, a1B])
    y = _cat4([y0, y1, y2, y3])
    h = jnp.maximum((agg + y) * dinv + b_ref[...], 0.0)
    yn = jnp.dot(h, w_ref[...]) * dinv
    _split_quarters(o_ref, yn)


def _layer(aggquart, yquart, dinv, b, w_next):
    return pl.pallas_call(
        _layer_body,
        grid=(_NB,),
        in_specs=(
            _quarter_specs() + _quarter_specs() + [
                pl.BlockSpec((_BN, 1), lambda i: (i, 0)),
                pl.BlockSpec((1, _H), lambda i: (0, 0)),
                pl.BlockSpec((_H, _H), lambda i: (0, 0)),
            ]
        ),
        out_specs=pl.BlockSpec((4, _BN, 16), lambda i: (0, i, 0)),
        out_shape=jax.ShapeDtypeStruct((4, _N, 16), jnp.float32),
    )(aggquart, aggquart, aggquart, aggquart, yquart, yquart, yquart,
      yquart, dinv, b, w_next)


def _final_body(a0A, a0B, a1A, a1B, y0, y1, y2, y3, dinv_ref, b_ref,
                batch_ref, f1w_ref, f1b_ref, f2w_ref, f2b_ref, ow_ref,
                ob_ref, out_ref, sums, counts):
    i = pl.program_id(0)

    @pl.when(i == 0)
    def _():
        sums[...] = jnp.zeros_like(sums)
        counts[...] = jnp.zeros_like(counts)

    dinv = dinv_ref[...]
    agg = _cat4([a0A, a0B, a1A, a1B])
    y = _cat4([y0, y1, y2, y3])
    h = jnp.maximum((agg + y) * dinv + b_ref[...], 0.0)
    p = (batch_ref[...] == lax.broadcasted_iota(jnp.int32, (1, _G), 1))
    p = p.astype(jnp.float32)
    dn = (((0,), (0,)), ((), ()))
    sums[...] += lax.dot_general(p, h, dn, precision=_HIGH)
    counts[...] += lax.dot_general(p, jnp.ones((_BN, 1), jnp.float32), dn,
                                   precision=_HIGH)

    @pl.when(i == _NB - 1)
    def _():
        pooled = sums[...] / jnp.maximum(counts[...], 1.0)
        a1 = jnp.maximum(jnp.dot(pooled, f1w_ref[...]) + f1b_ref[...], 0.0)
        a2 = jnp.maximum(jnp.dot(a1, f2w_ref[...]) + f2b_ref[...], 0.0)
        out_ref[...] = jnp.dot(a2, ow_ref[...]) + ob_ref[...]


def _final(aggquart, yquart, dinv, b2, batch2d, f1w, f1b, f2w, f2b,
           ow, ob):
    return pl.pallas_call(
        _final_body,
        grid=(_NB,),
        in_specs=(
            _quarter_specs() + _quarter_specs() + [
                pl.BlockSpec((_BN, 1), lambda i: (i, 0)),
                pl.BlockSpec((1, _H), lambda i: (0, 0)),
                pl.BlockSpec((_BN, 1), lambda i: (i, 0)),
                pl.BlockSpec((_H, 32), lambda i: (0, 0)),
                pl.BlockSpec((1, 32), lambda i: (0, 0)),
                pl.BlockSpec((32, 16), lambda i: (0, 0)),
                pl.BlockSpec((1, 16), lambda i: (0, 0)),
                pl.BlockSpec((16, 1), lambda i: (0, 0)),
                pl.BlockSpec((1, 1), lambda i: (0, 0)),
            ]
        ),
        out_specs=pl.BlockSpec((_G, 1), lambda i: (0, 0)),
        out_shape=jax.ShapeDtypeStruct((_G, 1), jnp.float32),
        scratch_shapes=[
            pltpu.VMEM((_G, _H), jnp.float32),
            pltpu.VMEM((_G, 1), jnp.float32),
        ],
    )(aggquart, aggquart, aggquart, aggquart, yquart, yquart, yquart,
      yquart, dinv, b2, batch2d, f1w, f1b, f2w, f2b, ow, ob)


def kernel(x, edge_index, batch, node_emb, conv_W0, conv_b0, conv_W1,
           conv_b1, conv_W2, conv_b2, fc1_W, fc1_b, fc2_W, fc2_b,
           out_W, out_b):
    x2d = x.astype(jnp.int32)
    src = edge_index[0].astype(jnp.int32)
    dst = edge_index[1].astype(jnp.int32)
    pad = _EPAD - _E
    src2d = jnp.concatenate(
        [src, jnp.zeros((pad,), jnp.int32)]).reshape(_NCH, 128)
    dst2d = jnp.concatenate(
        [dst, jnp.full((pad,), _TRASH, jnp.int32)]).reshape(_NCH, 128)

    one_pass_n = jnp.full((16,), 1, jnp.int32)
    two_pass_n = jnp.full((16,), 2, jnp.int32)
    ones_q = jnp.ones((4 * _N, 16), jnp.float32)
    degflat = _edge_pass(ones_q, src2d, dst2d, one_pass_n)
    y, dinv = _prep(x2d, degflat, node_emb, conv_W0)

    b0 = conv_b0.reshape(1, _H)
    b1 = conv_b1.reshape(1, _H)
    b2 = conv_b2.reshape(1, _H)

    yq = y.reshape(4 * _N, 16)
    agg = _edge_pass(yq, src2d, dst2d, two_pass_n)
    y = _layer(agg, yq, dinv, b0, conv_W1)

    yq = y.reshape(4 * _N, 16)
    agg = _edge_pass(yq, src2d, dst2d, two_pass_n)
    y = _layer(agg, yq, dinv, b1, conv_W2)

    yq = y.reshape(4 * _N, 16)
    agg = _edge_pass(yq, src2d, dst2d, two_pass_n)
    out = _final(agg, yq, dinv, b2,
                 batch.reshape(_N, 1).astype(jnp.int32),
                 fc1_W, fc1_b.reshape(1, 32), fc2_W, fc2_b.reshape(1, 16),
                 out_W, out_b.reshape(1, 1))
    return out[:, 0]


# speedup vs baseline: 11.9220x; 1.0483x over previous
import jax
import jax.numpy as jnp
from jax import lax
from jax.experimental import pallas as pl
from jax.experimental.pallas import tpu as pltpu
from jax.experimental.pallas import tpu_sc as plsc

_N = 50000
_E = 800000
_H = 64
_G = 512
_EPAD = 819200
_NCH = _EPAD // 128
_SROWS = 51200
_TRASH = _N
_ZSPAN = _SROWS // 16
_ZBUF = 800
_BCH = 40
_FIRE = 16

_HIGH = lax.Precision.HIGHEST


def _sc_mesh():
    return plsc.VectorSubcoreMesh(core_axis_name="c", subcore_axis_name="s")


def _edge_kernel(y_hbm, s_hbm, d_hbm, np_hbm, out_hbm, sidx, didx, rows,
                 zbuf, npv, agg, sem, sem2):
    c = lax.axis_index("c")
    s = lax.axis_index("s")
    z16 = jnp.zeros((16,), jnp.float32)
    ch0 = s * (_NCH // 16)
    pltpu.sync_copy(np_hbm, npv)
    npasses = npv[...][0]

    def one_pass(p, carry0):
        q = 2 * p + c
        qoff = jnp.full((16,), q * _N, jnp.int32)

        def zrow(i, carry):
            zbuf[i, pl.ds(0, 16)] = z16
            return carry
        lax.fori_loop(0, _ZBUF, zrow, 0)

        def zcp(i, carry):
            pltpu.sync_copy(zbuf, agg.at[pl.ds(s * _ZSPAN + i * _ZBUF,
                                               _ZBUF)])
            return carry
        lax.fori_loop(0, _ZSPAN // _ZBUF, zcp, 0)
        plsc.subcore_barrier()

        def batch(jb, carry):
            row0 = ch0 + jb * _BCH
            pltpu.sync_copy(s_hbm.at[pl.ds(row0, _BCH)], sidx)
            pltpu.sync_copy(d_hbm.at[pl.ds(row0, _BCH)], didx)

            def adjust(j, carry2):
                for g in range(8):
                    sidx[j, pl.ds(g * 16, 16)] = (
                        sidx[j, pl.ds(g * 16, 16)] + qoff)
                return carry2
            lax.fori_loop(0, _BCH, adjust, 0)

            gd = [None] * _BCH
            sd = [None] * _BCH
            lag = _FIRE // 2
            for j in range(_BCH + lag):
                if j < _BCH:
                    if j >= _FIRE:
                        sd[j - _FIRE].wait()
                    gd[j] = pltpu.async_copy(y_hbm.at[sidx.at[j]],
                                             rows.at[j % _FIRE], sem)
                if j >= lag and j - lag < _BCH:
                    k = j - lag
                    gd[k].wait()
                    sd[k] = pltpu.async_copy(rows.at[k % _FIRE],
                                             agg.at[didx.at[k]], sem2,
                                             add=True)
            for k in range(_BCH - _FIRE, _BCH):
                sd[k].wait()
            return carry
        lax.fori_loop(0, (_NCH // 16) // _BCH, batch, 0)
        plsc.subcore_barrier()

        base_sp = s * _ZSPAN
        base_out = q * _N + s * _ZSPAN

        @pl.when(s < 15)
        def _():
            def wb(i, carry):
                pltpu.sync_copy(agg.at[pl.ds(base_sp + i * _ZBUF, _ZBUF)],
                                zbuf)
                pltpu.sync_copy(zbuf, out_hbm.at[pl.ds(base_out + i * _ZBUF,
                                                       _ZBUF)])
                return carry
            lax.fori_loop(0, _ZSPAN // _ZBUF, wb, 0)

        @pl.when(s == 15)
        def _():
            def wb(i, carry):
                pltpu.sync_copy(agg.at[pl.ds(base_sp + i * 400, 400)],
                                zbuf.at[pl.ds(0, 400)])
                pltpu.sync_copy(zbuf.at[pl.ds(0, 400)],
                                out_hbm.at[pl.ds(base_out + i * 400, 400)])
                return carry
            lax.fori_loop(0, (_N - 15 * _ZSPAN) // 400, wb, 0)
        return carry0
    lax.fori_loop(0, npasses, one_pass, 0)


def _edge_pass(yquart, src2d, dst2d, npasses):
    return pl.kernel(
        _edge_kernel,
        out_type=jax.ShapeDtypeStruct((4 * _N, 16), jnp.float32),
        mesh=_sc_mesh(),
        scratch_types=[
            pltpu.VMEM((_BCH, 128), jnp.int32),
            pltpu.VMEM((_BCH, 128), jnp.int32),
            pltpu.VMEM((_FIRE, 128, 16), jnp.float32),
            pltpu.VMEM((_ZBUF, 16), jnp.float32),
            pltpu.VMEM((16,), jnp.int32),
            pltpu.VMEM_SHARED((_SROWS, 16), jnp.float32),
            pltpu.SemaphoreType.DMA,
            pltpu.SemaphoreType.DMA,
        ],
        compiler_params=pltpu.CompilerParams(use_tc_tiling_on_sc=False),
    )(yquart, src2d, dst2d, npasses)


_BN = 1000
_NB = _N // _BN


def _quarter_specs():
    return [pl.BlockSpec((_BN, 16), lambda i, q=q: (q * _NB + i, 0))
            for q in range(4)]


def _cat4(refs):
    return jnp.concatenate([r[...] for r in refs], axis=1)


def _split_quarters(y_ref, y):
    for q in range(4):
        y_ref[q] = y[:, q * 16:(q + 1) * 16]


def _prep_body(x_ref, deg_ref, emb_ref, w_ref, y_ref, dinv_ref):
    deg = deg_ref[:, 0] + 1.0
    dinv = 1.0 / jnp.sqrt(deg)
    dinv_ref[...] = dinv[:, None]
    onehot = (x_ref[...] == lax.broadcasted_iota(jnp.int32, (1, 100), 1))
    h0 = jnp.dot(onehot.astype(jnp.float32), emb_ref[...], precision=_HIGH)
    y = jnp.dot(h0, w_ref[...]) * dinv[:, None]
    _split_quarters(y_ref, y)


def _prep(x2d, degflat, node_emb, w0):
    return pl.pallas_call(
        _prep_body,
        grid=(_NB,),
        in_specs=[
            pl.BlockSpec((_BN, 1), lambda i: (i, 0)),
            pl.BlockSpec((_BN, 16), lambda i: (i, 0)),
            pl.BlockSpec((100, _H), lambda i: (0, 0)),
            pl.BlockSpec((_H, _H), lambda i: (0, 0)),
        ],
        out_specs=[
            pl.BlockSpec((4, _BN, 16), lambda i: (0, i, 0)),
            pl.BlockSpec((_BN, 1), lambda i: (i, 0)),
        ],
        out_shape=[
            jax.ShapeDtypeStruct((4, _N, 16), jnp.float32),
            jax.ShapeDtypeStruct((_N, 1), jnp.float32),
        ],
    )(x2d, degflat, node_emb, w0)


def _layer_body(a0A, a0B, a1A, a1B, y0, y1, y2, y3, dinv_ref, b_ref, w_ref,
                o_ref):
    dinv = dinv_ref[...]
    agg = _cat4([a0A, a0B, a1A, a1B])
    y = _cat4([y0, y1, y2, y3])
    h = jnp.maximum((agg + y) * dinv + b_ref[...], 0.0)
    yn = jnp.dot(h, w_ref[...]) * dinv
    _split_quarters(o_ref, yn)


def _layer(aggquart, yquart, dinv, b, w_next):
    return pl.pallas_call(
        _layer_body,
        grid=(_NB,),
        in_specs=(
            _quarter_specs() + _quarter_specs() + [
                pl.BlockSpec((_BN, 1), lambda i: (i, 0)),
                pl.BlockSpec((1, _H), lambda i: (0, 0)),
                pl.BlockSpec((_H, _H), lambda i: (0, 0)),
            ]
        ),
        out_specs=pl.BlockSpec((4, _BN, 16), lambda i: (0, i, 0)),
        out_shape=jax.ShapeDtypeStruct((4, _N, 16), jnp.float32),
    )(aggquart, aggquart, aggquart, aggquart, yquart, yquart, yquart,
      yquart, dinv, b, w_next)


def _final_body(a0A, a0B, a1A, a1B, y0, y1, y2, y3, dinv_ref, b_ref,
                batch_ref, f1w_ref, f1b_ref, f2w_ref, f2b_ref, ow_ref,
                ob_ref, out_ref, sums, counts):
    i = pl.program_id(0)

    @pl.when(i == 0)
    def _():
        sums[...] = jnp.zeros_like(sums)
        counts[...] = jnp.zeros_like(counts)

    dinv = dinv_ref[...]
    agg = _cat4([a0A, a0B, a1A, a1B])
    y = _cat4([y0, y1, y2, y3])
    h = jnp.maximum((agg + y) * dinv + b_ref[...], 0.0)
    p = (batch_ref[...] == lax.broadcasted_iota(jnp.int32, (1, _G), 1))
    p = p.astype(jnp.float32)
    dn = (((0,), (0,)), ((), ()))
    sums[...] += lax.dot_general(p, h, dn, precision=_HIGH)
    counts[...] += lax.dot_general(p, jnp.ones((_BN, 1), jnp.float32), dn,
                                   precision=_HIGH)

    @pl.when(i == _NB - 1)
    def _():
        pooled = sums[...] / jnp.maximum(counts[...], 1.0)
        a1 = jnp.maximum(jnp.dot(pooled, f1w_ref[...]) + f1b_ref[...], 0.0)
        a2 = jnp.maximum(jnp.dot(a1, f2w_ref[...]) + f2b_ref[...], 0.0)
        out_ref[...] = jnp.dot(a2, ow_ref[...]) + ob_ref[...]


def _final(aggquart, yquart, dinv, b2, batch2d, f1w, f1b, f2w, f2b,
           ow, ob):
    return pl.pallas_call(
        _final_body,
        grid=(_NB,),
        in_specs=(
            _quarter_specs() + _quarter_specs() + [
                pl.BlockSpec((_BN, 1), lambda i: (i, 0)),
                pl.BlockSpec((1, _H), lambda i: (0, 0)),
                pl.BlockSpec((_BN, 1), lambda i: (i, 0)),
                pl.BlockSpec((_H, 32), lambda i: (0, 0)),
                pl.BlockSpec((1, 32), lambda i: (0, 0)),
                pl.BlockSpec((32, 16), lambda i: (0, 0)),
                pl.BlockSpec((1, 16), lambda i: (0, 0)),
                pl.BlockSpec((16, 1), lambda i: (0, 0)),
                pl.BlockSpec((1, 1), lambda i: (0, 0)),
            ]
        ),
        out_specs=pl.BlockSpec((_G, 1), lambda i: (0, 0)),
        out_shape=jax.ShapeDtypeStruct((_G, 1), jnp.float32),
        scratch_shapes=[
            pltpu.VMEM((_G, _H), jnp.float32),
            pltpu.VMEM((_G, 1), jnp.float32),
        ],
    )(aggquart, aggquart, aggquart, aggquart, yquart, yquart, yquart,
      yquart, dinv, b2, batch2d, f1w, f1b, f2w, f2b, ow, ob)


def kernel(x, edge_index, batch, node_emb, conv_W0, conv_b0, conv_W1,
           conv_b1, conv_W2, conv_b2, fc1_W, fc1_b, fc2_W, fc2_b,
           out_W, out_b):
    x2d = x.astype(jnp.int32)
    src = edge_index[0].astype(jnp.int32)
    dst = edge_index[1].astype(jnp.int32)
    pad = _EPAD - _E
    src2d = jnp.concatenate(
        [src, jnp.zeros((pad,), jnp.int32)]).reshape(_NCH, 128)
    dst2d = jnp.concatenate(
        [dst, jnp.full((pad,), _TRASH, jnp.int32)]).reshape(_NCH, 128)

    one_pass_n = jnp.full((16,), 1, jnp.int32)
    two_pass_n = jnp.full((16,), 2, jnp.int32)
    ones_q = jnp.ones((4 * _N, 16), jnp.float32)
    degflat = _edge_pass(ones_q, src2d, dst2d, one_pass_n)
    y, dinv = _prep(x2d, degflat, node_emb, conv_W0)

    b0 = conv_b0.reshape(1, _H)
    b1 = conv_b1.reshape(1, _H)
    b2 = conv_b2.reshape(1, _H)

    yq = y.reshape(4 * _N, 16)
    agg = _edge_pass(yq, src2d, dst2d, two_pass_n)
    y = _layer(agg, yq, dinv, b0, conv_W1)

    yq = y.reshape(4 * _N, 16)
    agg = _edge_pass(yq, src2d, dst2d, two_pass_n)
    y = _layer(agg, yq, dinv, b1, conv_W2)

    yq = y.reshape(4 * _N, 16)
    agg = _edge_pass(yq, src2d, dst2d, two_pass_n)
    out = _final(agg, yq, dinv, b2,
                 batch.reshape(_N, 1).astype(jnp.int32),
                 fc1_W, fc1_b.reshape(1, 32), fc2_W, fc2_b.reshape(1, 16),
                 out_W, out_b.reshape(1, 1))
    return out[:, 0]


# speedup vs baseline: 12.1568x; 1.0197x over previous
import jax
import jax.numpy as jnp
from jax import lax
from jax.experimental import pallas as pl
from jax.experimental.pallas import tpu as pltpu
from jax.experimental.pallas import tpu_sc as plsc

_N = 50000
_E = 800000
_H = 64
_G = 512
_EPAD = 819200
_NCH = _EPAD // 128
_SROWS = 51200
_TRASH = _N
_ZSPAN = _SROWS // 16
_ZBUF = 800
_BCH = 80
_FIRE = 16

_HIGH = lax.Precision.HIGHEST


def _sc_mesh():
    return plsc.VectorSubcoreMesh(core_axis_name="c", subcore_axis_name="s")


def _edge_kernel(y_hbm, s_hbm, d_hbm, np_hbm, out_hbm, sidx, didx, rows,
                 zbuf, npv, agg, sem, sem2):
    c = lax.axis_index("c")
    s = lax.axis_index("s")
    z16 = jnp.zeros((16,), jnp.float32)
    ch0 = s * (_NCH // 16)
    pltpu.sync_copy(np_hbm, npv)
    npasses = npv[...][0]

    def one_pass(p, carry0):
        q = 2 * p + c
        qoff = jnp.full((16,), q * _N, jnp.int32)

        def zrow(i, carry):
            zbuf[i, pl.ds(0, 16)] = z16
            return carry
        lax.fori_loop(0, _ZBUF, zrow, 0)

        def zcp(i, carry):
            pltpu.sync_copy(zbuf, agg.at[pl.ds(s * _ZSPAN + i * _ZBUF,
                                               _ZBUF)])
            return carry
        lax.fori_loop(0, _ZSPAN // _ZBUF, zcp, 0)
        plsc.subcore_barrier()

        def batch(jb, carry):
            row0 = ch0 + jb * _BCH
            pltpu.sync_copy(s_hbm.at[pl.ds(row0, _BCH)], sidx)
            pltpu.sync_copy(d_hbm.at[pl.ds(row0, _BCH)], didx)

            def adjust(j, carry2):
                for g in range(8):
                    sidx[j, pl.ds(g * 16, 16)] = (
                        sidx[j, pl.ds(g * 16, 16)] + qoff)
                return carry2
            lax.fori_loop(0, _BCH, adjust, 0)

            gd = [None] * _BCH
            sd = [None] * _BCH
            lag = _FIRE // 2
            for j in range(_BCH + lag):
                if j < _BCH:
                    if j >= _FIRE:
                        sd[j - _FIRE].wait()
                    gd[j] = pltpu.async_copy(y_hbm.at[sidx.at[j]],
                                             rows.at[j % _FIRE], sem)
                if j >= lag and j - lag < _BCH:
                    k = j - lag
                    gd[k].wait()
                    sd[k] = pltpu.async_copy(rows.at[k % _FIRE],
                                             agg.at[didx.at[k]], sem2,
                                             add=True)
            for k in range(_BCH - _FIRE, _BCH):
                sd[k].wait()
            return carry
        lax.fori_loop(0, (_NCH // 16) // _BCH, batch, 0)
        plsc.subcore_barrier()

        base_sp = s * _ZSPAN
        base_out = q * _N + s * _ZSPAN

        @pl.when(s < 15)
        def _():
            def wb(i, carry):
                pltpu.sync_copy(agg.at[pl.ds(base_sp + i * _ZBUF, _ZBUF)],
                                zbuf)
                pltpu.sync_copy(zbuf, out_hbm.at[pl.ds(base_out + i * _ZBUF,
                                                       _ZBUF)])
                return carry
            lax.fori_loop(0, _ZSPAN // _ZBUF, wb, 0)

        @pl.when(s == 15)
        def _():
            def wb(i, carry):
                pltpu.sync_copy(agg.at[pl.ds(base_sp + i * 400, 400)],
                                zbuf.at[pl.ds(0, 400)])
                pltpu.sync_copy(zbuf.at[pl.ds(0, 400)],
                                out_hbm.at[pl.ds(base_out + i * 400, 400)])
                return carry
            lax.fori_loop(0, (_N - 15 * _ZSPAN) // 400, wb, 0)
        return carry0
    lax.fori_loop(0, npasses, one_pass, 0)


def _edge_pass(yquart, src2d, dst2d, npasses):
    return pl.kernel(
        _edge_kernel,
        out_type=jax.ShapeDtypeStruct((4 * _N, 16), jnp.float32),
        mesh=_sc_mesh(),
        scratch_types=[
            pltpu.VMEM((_BCH, 128), jnp.int32),
            pltpu.VMEM((_BCH, 128), jnp.int32),
            pltpu.VMEM((_FIRE, 128, 16), jnp.float32),
            pltpu.VMEM((_ZBUF, 16), jnp.float32),
            pltpu.VMEM((16,), jnp.int32),
            pltpu.VMEM_SHARED((_SROWS, 16), jnp.float32),
            pltpu.SemaphoreType.DMA,
            pltpu.SemaphoreType.DMA,
        ],
        compiler_params=pltpu.CompilerParams(use_tc_tiling_on_sc=False),
    )(yquart, src2d, dst2d, npasses)


_BN = 1000
_NB = _N // _BN


def _quarter_specs():
    return [pl.BlockSpec((_BN, 16), lambda i, q=q: (q * _NB + i, 0))
            for q in range(4)]


def _cat4(refs):
    return jnp.concatenate([r[...] for r in refs], axis=1)


def _split_quarters(y_ref, y):
    for q in range(4):
        y_ref[q] = y[:, q * 16:(q + 1) * 16]


def _prep_body(x_ref, deg_ref, emb_ref, w_ref, y_ref, dinv_ref):
    deg = deg_ref[:, 0] + 1.0
    dinv = 1.0 / jnp.sqrt(deg)
    dinv_ref[...] = dinv[:, None]
    onehot = (x_ref[...] == lax.broadcasted_iota(jnp.int32, (1, 100), 1))
    h0 = jnp.dot(onehot.astype(jnp.float32), emb_ref[...], precision=_HIGH)
    y = jnp.dot(h0, w_ref[...]) * dinv[:, None]
    _split_quarters(y_ref, y)


def _prep(x2d, degflat, node_emb, w0):
    return pl.pallas_call(
        _prep_body,
        grid=(_NB,),
        in_specs=[
            pl.BlockSpec((_BN, 1), lambda i: (i, 0)),
            pl.BlockSpec((_BN, 16), lambda i: (i, 0)),
            pl.BlockSpec((100, _H), lambda i: (0, 0)),
            pl.BlockSpec((_H, _H), lambda i: (0, 0)),
        ],
        out_specs=[
            pl.BlockSpec((4, _BN, 16), lambda i: (0, i, 0)),
            pl.BlockSpec((_BN, 1), lambda i: (i, 0)),
        ],
        out_shape=[
            jax.ShapeDtypeStruct((4, _N, 16), jnp.float32),
            jax.ShapeDtypeStruct((_N, 1), jnp.float32),
        ],
    )(x2d, degflat, node_emb, w0)


def _layer_body(a0A, a0B, a1A, a1B, y0, y1, y2, y3, dinv_ref, b_ref, w_ref,
                o_ref):
    dinv = dinv_ref[...]
    agg = _cat4([a0A, a0B, a1A, a1B])
    y = _cat4([y0, y1, y2, y3])
    h = jnp.maximum((agg + y) * dinv + b_ref[...], 0.0)
    yn = jnp.dot(h, w_ref[...]) * dinv
    _split_quarters(o_ref, yn)


def _layer(aggquart, yquart, dinv, b, w_next):
    return pl.pallas_call(
        _layer_body,
        grid=(_NB,),
        in_specs=(
            _quarter_specs() + _quarter_specs() + [
                pl.BlockSpec((_BN, 1), lambda i: (i, 0)),
                pl.BlockSpec((1, _H), lambda i: (0, 0)),
                pl.BlockSpec((_H, _H), lambda i: (0, 0)),
            ]
        ),
        out_specs=pl.BlockSpec((4, _BN, 16), lambda i: (0, i, 0)),
        out_shape=jax.ShapeDtypeStruct((4, _N, 16), jnp.float32),
    )(aggquart, aggquart, aggquart, aggquart, yquart, yquart, yquart,
      yquart, dinv, b, w_next)


def _final_body(a0A, a0B, a1A, a1B, y0, y1, y2, y3, dinv_ref, b_ref,
                batch_ref, f1w_ref, f1b_ref, f2w_ref, f2b_ref, ow_ref,
                ob_ref, out_ref, sums, counts):
    i = pl.program_id(0)

    @pl.when(i == 0)
    def _():
        sums[...] = jnp.zeros_like(sums)
        counts[...] = jnp.zeros_like(counts)

    dinv = dinv_ref[...]
    agg = _cat4([a0A, a0B, a1A, a1B])
    y = _cat4([y0, y1, y2, y3])
    h = jnp.maximum((agg + y) * dinv + b_ref[...], 0.0)
    p = (batch_ref[...] == lax.broadcasted_iota(jnp.int32, (1, _G), 1))
    p = p.astype(jnp.float32)
    dn = (((0,), (0,)), ((), ()))
    sums[...] += lax.dot_general(p, h, dn, precision=_HIGH)
    counts[...] += lax.dot_general(p, jnp.ones((_BN, 1), jnp.float32), dn,
                                   precision=_HIGH)

    @pl.when(i == _NB - 1)
    def _():
        pooled = sums[...] / jnp.maximum(counts[...], 1.0)
        a1 = jnp.maximum(jnp.dot(pooled, f1w_ref[...]) + f1b_ref[...], 0.0)
        a2 = jnp.maximum(jnp.dot(a1, f2w_ref[...]) + f2b_ref[...], 0.0)
        out_ref[...] = jnp.dot(a2, ow_ref[...]) + ob_ref[...]


def _final(aggquart, yquart, dinv, b2, batch2d, f1w, f1b, f2w, f2b,
           ow, ob):
    return pl.pallas_call(
        _final_body,
        grid=(_NB,),
        in_specs=(
            _quarter_specs() + _quarter_specs() + [
                pl.BlockSpec((_BN, 1), lambda i: (i, 0)),
                pl.BlockSpec((1, _H), lambda i: (0, 0)),
                pl.BlockSpec((_BN, 1), lambda i: (i, 0)),
                pl.BlockSpec((_H, 32), lambda i: (0, 0)),
                pl.BlockSpec((1, 32), lambda i: (0, 0)),
                pl.BlockSpec((32, 16), lambda i: (0, 0)),
                pl.BlockSpec((1, 16), lambda i: (0, 0)),
                pl.BlockSpec((16, 1), lambda i: (0, 0)),
                pl.BlockSpec((1, 1), lambda i: (0, 0)),
            ]
        ),
        out_specs=pl.BlockSpec((_G, 1), lambda i: (0, 0)),
        out_shape=jax.ShapeDtypeStruct((_G, 1), jnp.float32),
        scratch_shapes=[
            pltpu.VMEM((_G, _H), jnp.float32),
            pltpu.VMEM((_G, 1), jnp.float32),
        ],
    )(aggquart, aggquart, aggquart, aggquart, yquart, yquart, yquart,
      yquart, dinv, b2, batch2d, f1w, f1b, f2w, f2b, ow, ob)


def kernel(x, edge_index, batch, node_emb, conv_W0, conv_b0, conv_W1,
           conv_b1, conv_W2, conv_b2, fc1_W, fc1_b, fc2_W, fc2_b,
           out_W, out_b):
    x2d = x.astype(jnp.int32)
    src = edge_index[0].astype(jnp.int32)
    dst = edge_index[1].astype(jnp.int32)
    pad = _EPAD - _E
    src2d = jnp.concatenate(
        [src, jnp.zeros((pad,), jnp.int32)]).reshape(_NCH, 128)
    dst2d = jnp.concatenate(
        [dst, jnp.full((pad,), _TRASH, jnp.int32)]).reshape(_NCH, 128)

    one_pass_n = jnp.full((16,), 1, jnp.int32)
    two_pass_n = jnp.full((16,), 2, jnp.int32)
    ones_q = jnp.ones((4 * _N, 16), jnp.float32)
    degflat = _edge_pass(ones_q, src2d, dst2d, one_pass_n)
    y, dinv = _prep(x2d, degflat, node_emb, conv_W0)

    b0 = conv_b0.reshape(1, _H)
    b1 = conv_b1.reshape(1, _H)
    b2 = conv_b2.reshape(1, _H)

    yq = y.reshape(4 * _N, 16)
    agg = _edge_pass(yq, src2d, dst2d, two_pass_n)
    y = _layer(agg, yq, dinv, b0, conv_W1)

    yq = y.reshape(4 * _N, 16)
    agg = _edge_pass(yq, src2d, dst2d, two_pass_n)
    y = _layer(agg, yq, dinv, b1, conv_W2)

    yq = y.reshape(4 * _N, 16)
    agg = _edge_pass(yq, src2d, dst2d, two_pass_n)
    out = _final(agg, yq, dinv, b2,
                 batch.reshape(_N, 1).astype(jnp.int32),
                 fc1_W, fc1_b.reshape(1, 32), fc2_W, fc2_b.reshape(1, 16),
                 out_W, out_b.reshape(1, 1))
    return out[:, 0]
